# baseline (speedup 1.0000x reference)
import functools
import math

import jax
import jax.numpy as jnp
from jax import lax
from jax.experimental import pallas as pl
from jax.experimental.pallas import tpu as pltpu
from jax.experimental.pallas import tpu_sc as plsc

B, N = 32, 32
NUM_BASIS = 20
R_MAX = 10.0
D_EMB = 32
D_SCAL = 64
_C = 1.0 / math.sqrt(D_EMB)
NC, NS, L = 2, 16, 16
BPS = 8
P = BPS * N * N


def _tc_body(pos_ref, fc1wT_ref, fc1b_ref, gamma_ref, ae_exp_ref, fc2w1_ref,
             fc2b1_ref, wself_ref, wread_ref, atom_ref,
             hid_ref, vt_ref):
    @pl.when(pl.program_id(0) == 0)
    def _():
        v = jnp.dot(ae_exp_ref[...], fc2w1_ref[...])
        s0 = jnp.dot(ae_exp_ref[...], fc2b1_ref[...])
        q = jnp.dot(wself_ref[...], wread_ref[...])
        aeq = jnp.dot(atom_ref[...], q) * _C
        w3 = (_C * (N - 1)) * s0 + aeq
        vt_ref[...] = jnp.concatenate([v, w3], axis=1)

    pos = pos_ref[...]
    pi = jnp.broadcast_to(pos[:, :, None, :], (BPS, N, N, 3)).reshape(P, 3)
    pj = jnp.broadcast_to(pos[:, None, :, :], (BPS, N, N, 3)).reshape(P, 3)
    diff = pi - pj
    d2 = jnp.sum(diff * diff, axis=1, keepdims=True)
    d = jnp.sqrt(jnp.maximum(d2, 0.0))
    centers = jax.lax.broadcasted_iota(jnp.int32, (1, NUM_BASIS), 1).astype(
        jnp.float32) * (R_MAX / (NUM_BASIS - 1))
    g = gamma_ref[...]
    t = d - centers
    rbf = jnp.exp(-g * t * t)
    hid = jnp.maximum(jnp.dot(rbf, fc1wT_ref[...]) + fc1b_ref[...], 0.0)
    hid_ref[...] = jnp.transpose(hid.reshape(BPS, N * N, D_EMB), (0, 2, 1))


def _zchunk(nav, c):
    a0 = nav[0, pl.ds(L * c, L)]
    a1 = nav[1, pl.ds(L * c, L)]
    a2 = nav[2, pl.ds(L * c, L)]
    one = jnp.full((L,), 1, jnp.int32)
    z = jnp.where(a1 > a0, one, jnp.zeros((L,), jnp.int32))
    z = jnp.where(a2 > jnp.maximum(a0, a1), one + one, z)
    return z


def _zchunk2(nav, c):
    jr = lax.iota(jnp.int32, L) + (L * c)
    a0 = plsc.load_gather(nav, [jr, jnp.zeros((L,), jnp.int32)])
    a1 = plsc.load_gather(nav, [jr, jnp.zeros((L,), jnp.int32) + 1])
    a2 = plsc.load_gather(nav, [jr, jnp.zeros((L,), jnp.int32) + 2])
    one = jnp.full((L,), 1, jnp.int32)
    z = jnp.where(a1 > a0, one, jnp.zeros((L,), jnp.int32))
    z = jnp.where(a2 > jnp.maximum(a0, a1), one + one, z)
    return z


def _sc_body(hid3, na, vt, out, hidv, nav, vtv, outv):
    wid = lax.axis_index("s") * NC + lax.axis_index("c")
    b_local = wid // 2
    c = wid % 2
    pltpu.sync_copy(hid3.at[b_local], hidv)
    pltpu.sync_copy(na.at[b_local], nav)
    pltpu.sync_copy(vt, vtv)

    z = _zchunk2(nav, c)
    total = plsc.load_gather(vtv, [z, jnp.full((L,), D_EMB, jnp.int32)])
    j_ids = lax.iota(jnp.int32, L) + (L * c)
    vzs = [plsc.load_gather(vtv, [z, jnp.full((L,), u, jnp.int32)])
           for u in range(D_EMB)]

    def body(i, pacc):
        acc_i = pacc
        for half in range(2):
            i2 = i + half * (N // 2)
            accs = [jnp.zeros((L,), jnp.float32) for _ in range(4)]
            for u in range(D_EMB):
                hu = hidv[u, pl.ds(i2 * N + L * c, L)]
                accs[u % 4] = accs[u % 4] + hu * vzs[u]
            acc = (accs[0] + accs[1]) + (accs[2] + accs[3])
            acc_i = acc_i + jnp.where(j_ids != i2, acc, 0.0)
        return acc_i

    pair = lax.fori_loop(0, N // 2, body, jnp.zeros((L,), jnp.float32))
    total = total + _C * pair

    tot = jnp.sum(0.125 * total)
    outv[...] = jnp.broadcast_to(tot, (L,))
    pltpu.sync_copy(outv, out.at[b_local, c])


HB = B // 2


def _half_chain(pos_h, na_h, fc1wT, fc1b, gamma2, ae_exp, fc2w1, fc2b1,
                w_self, w_readout, atom_emb):
    grid = (HB // BPS,)
    full = lambda shape: pl.BlockSpec(shape, lambda b: (0,) * len(shape))
    hid, vt = pl.pallas_call(
        _tc_body,
        grid=grid,
        in_specs=[
            pl.BlockSpec((BPS, N, 3), lambda b: (b, 0, 0)),
            full((NUM_BASIS, 32)),
            full((1, 32)),
            full((1, 1)),
            full((3, D_EMB * D_SCAL)),
            full((D_EMB * D_SCAL, 32)),
            full((D_EMB * D_SCAL, 1)),
            full((D_EMB, D_SCAL)),
            full((D_SCAL, 1)),
            full((3, D_EMB)),
        ],
        out_specs=[
            pl.BlockSpec((BPS, D_EMB, N * N), lambda b: (b, 0, 0)),
            full((3, D_EMB + 1)),
        ],
        out_shape=[
            jax.ShapeDtypeStruct((HB, D_EMB, N * N), jnp.float32),
            jax.ShapeDtypeStruct((3, D_EMB + 1), jnp.float32),
        ],
    )(pos_h, fc1wT, fc1b, gamma2, ae_exp, fc2w1, fc2b1, w_self, w_readout,
      atom_emb)

    mesh = plsc.VectorSubcoreMesh(core_axis_name="c", subcore_axis_name="s")
    sc = functools.partial(
        pl.kernel,
        mesh=mesh,
        compiler_params=pltpu.CompilerParams(needs_layout_passes=False),
        out_type=jax.ShapeDtypeStruct((HB, 2, L), jnp.float32),
        scratch_types=[
            pltpu.VMEM((D_EMB, N * N), jnp.float32),
            pltpu.VMEM((N, 3), jnp.float32),
            pltpu.VMEM((3, D_EMB + 1), jnp.float32),
            pltpu.VMEM((L,), jnp.float32),
        ],
    )(_sc_body)
    out = sc(hid, na_h, vt)
    return out[:, 0, 0:1] + out[:, 1, 0:1]


def kernel(pos, node_attrs, atom_emb, gamma, fc1_w, fc1_b, fc2_w, fc2_b,
           w_self, w_readout):
    ae_exp = (atom_emb[:, :, None] * w_readout[None, None, :, 0]).reshape(
        3, D_EMB * D_SCAL)
    fc2w1 = fc2_w[:D_EMB * D_SCAL, :]
    fc2b1 = fc2_b[:D_EMB * D_SCAL].reshape(D_EMB * D_SCAL, 1)
    fc1wT = fc1_w.T
    fc1b = fc1_b.reshape(1, 32)
    gamma2 = jnp.asarray(gamma, jnp.float32).reshape(1, 1)

    args = (fc1wT, fc1b, gamma2, ae_exp, fc2w1, fc2b1, w_self, w_readout,
            atom_emb)
    t1 = _half_chain(pos[:HB], node_attrs[:HB], *args)
    t2 = _half_chain(pos[HB:], node_attrs[HB:], *args)
    return jnp.concatenate([t1, t2], axis=0)
